# baseline (speedup 1.0000x reference)
import jax
import jax.numpy as jnp
from jax import lax
from jax.experimental import pallas as pl
from jax.experimental.pallas import tpu as pltpu

_IPB = 2
_NBUF = 4


def _cc_body(cam_ref, w_ref, b_ref, tex_hbm, out_hbm,
             in_buf, out_buf, in_sems, out_sems):
    n_step = tex_hbm.shape[0] // _IPB

    def in_copy(step, slot):
        return pltpu.make_async_copy(
            tex_hbm.at[pl.ds(step * _IPB, _IPB)],
            in_buf.at[slot], in_sems.at[slot])

    def out_copy(step, slot):
        return pltpu.make_async_copy(
            out_buf.at[slot],
            out_hbm.at[pl.ds(step * _IPB, _IPB)], out_sems.at[slot])

    for s in range(_NBUF):
        in_copy(s, s).start()

    def step_fn(step, carry):
        slot = lax.rem(step, _NBUF)
        in_copy(step, slot).wait()

        @pl.when(step >= _NBUF)
        def _():
            out_copy(step - _NBUF, slot).wait()

        for k in range(_IPB):
            ci = cam_ref[step * _IPB + k]
            for c in range(3):
                w = w_ref[ci, c]
                b = b_ref[ci, c]
                out_buf[slot, k, c] = in_buf[slot, k, c] * w + b

        out_copy(step, slot).start()

        @pl.when(step + _NBUF < n_step)
        def _():
            in_copy(step + _NBUF, slot).start()

        return carry

    lax.fori_loop(0, n_step, step_fn, 0)

    for s in range(_NBUF):
        step = n_step - _NBUF + s
        out_copy(step, step % _NBUF).wait()


@jax.jit
def kernel(texture, cam, weight, bias):
    B, C, H, W = texture.shape
    dt = texture.dtype
    w_full = jnp.concatenate(
        [jnp.ones((1, C), dt), weight.reshape(-1, C)], axis=0)
    b_full = jnp.concatenate(
        [jnp.zeros((1, C), dt), bias.reshape(-1, C)], axis=0)
    cam32 = cam.astype(jnp.int32)
    return pl.pallas_call(
        _cc_body,
        in_specs=[
            pl.BlockSpec(memory_space=pltpu.SMEM),
            pl.BlockSpec(memory_space=pltpu.SMEM),
            pl.BlockSpec(memory_space=pltpu.SMEM),
            pl.BlockSpec(memory_space=pl.ANY),
        ],
        out_specs=pl.BlockSpec(memory_space=pl.ANY),
        out_shape=jax.ShapeDtypeStruct(texture.shape, dt),
        scratch_shapes=[
            pltpu.VMEM((_NBUF, _IPB, C, H, W), dt),
            pltpu.VMEM((_NBUF, _IPB, C, H, W), dt),
            pltpu.SemaphoreType.DMA((_NBUF,)),
            pltpu.SemaphoreType.DMA((_NBUF,)),
        ],
    )(cam32, w_full, b_full, texture)


# speedup vs baseline: 1.0454x; 1.0454x over previous
import jax
import jax.numpy as jnp
from jax.experimental import pallas as pl
from jax.experimental.pallas import tpu as pltpu

_IPB = 4


def _cc_body(cam_ref, w_ref, b_ref, tex_ref, out_ref):
    i = pl.program_id(0)
    for k in range(_IPB):
        ci = cam_ref[i * _IPB + k]
        for c in range(3):
            w = w_ref[ci, c]
            b = b_ref[ci, c]
            out_ref[k, c] = tex_ref[k, c] * w + b


@jax.jit
def kernel(texture, cam, weight, bias):
    B, C, H, W = texture.shape
    dt = texture.dtype
    w_full = jnp.concatenate(
        [jnp.ones((1, C), dt), weight.reshape(-1, C)], axis=0)
    b_full = jnp.concatenate(
        [jnp.zeros((1, C), dt), bias.reshape(-1, C)], axis=0)
    cam32 = cam.astype(jnp.int32)
    return pl.pallas_call(
        _cc_body,
        grid=(B // _IPB,),
        in_specs=[
            pl.BlockSpec(memory_space=pltpu.SMEM),
            pl.BlockSpec(memory_space=pltpu.SMEM),
            pl.BlockSpec(memory_space=pltpu.SMEM),
            pl.BlockSpec((_IPB, C, H, W), lambda i: (i, 0, 0, 0)),
        ],
        out_specs=pl.BlockSpec((_IPB, C, H, W), lambda i: (i, 0, 0, 0)),
        out_shape=jax.ShapeDtypeStruct(texture.shape, dt),
        compiler_params=pltpu.CompilerParams(
            dimension_semantics=("arbitrary",)),
    )(cam32, w_full, b_full, texture)
